# baseline (speedup 1.0000x reference)
import functools

import jax
import jax.numpy as jnp
from jax import lax
from jax.experimental import pallas as pl
from jax.experimental.pallas import tpu as pltpu
from jax.experimental.pallas import tpu_sc as plsc

EPS = 1e-07
NC = 2
NS = 16
LANES = 16
K = 128


def _rsqrt_newton(v):
    i = plsc.bitcast(v, jnp.int32)
    y = plsc.bitcast(jnp.int32(0x5F3759DF) - (i >> 1), jnp.float32)
    for _ in range(3):
        y = y * (1.5 - 0.5 * v * y * y)
    return y


def _sc_coeff_stage(ncr, cpt, src_p, dst_p, posx, posy, posz):
    n = posx.shape[0]
    g_total = NC * NS * cpt
    mesh = plsc.VectorSubcoreMesh(core_axis_name="c", subcore_axis_name="s")

    @functools.partial(
        pl.kernel,
        out_type=(
            jax.ShapeDtypeStruct((g_total, 5, K), jnp.int32),
            jax.ShapeDtypeStruct((NC, ncr, K), jnp.float32),
        ),
        mesh=mesh,
        scratch_types=[
            pltpu.VMEM((n,), jnp.float32),
            pltpu.VMEM((n,), jnp.float32),
            pltpu.VMEM((n,), jnp.float32),
            pltpu.VMEM((2, K), jnp.int32),
            pltpu.VMEM((2, K), jnp.int32),
            pltpu.VMEM((5, K), jnp.int32),
            pltpu.VMEM((5, K), jnp.int32),
            pltpu.VMEM((ncr, K), jnp.float32),
            pltpu.VMEM((ncr,), jnp.int32),
            pltpu.VMEM_SHARED((ncr, K), jnp.float32),
            pltpu.SemaphoreType.DMA,
            pltpu.SemaphoreType.DMA,
            pltpu.SemaphoreType.DMA,
            pltpu.SemaphoreType.DMA,
        ],
        compiler_params=pltpu.CompilerParams(needs_layout_passes=False),
    )
    def coeff_kernel(src_hbm, dst_hbm, px_hbm, py_hbm, pz_hbm,
                     meta_hbm, cnt_hbm,
                     px_v, py_v, pz_v, in0, in1, out0, out1,
                     cnt_v, rowidx, cnt_sh, isem0, isem1, wsem0, wsem1):
        cid = lax.axis_index("c")
        sid = lax.axis_index("s")
        wid = cid * NS + sid

        pltpu.sync_copy(px_hbm, px_v)
        pltpu.sync_copy(py_hbm, py_v)
        pltpu.sync_copy(pz_hbm, pz_v)

        zero16 = jnp.zeros((LANES,), jnp.float32)
        iota16 = lax.iota(jnp.int32, LANES)

        def zero_cnt(r, _):
            for cc in range(K // LANES):
                cnt_v[r, pl.ds(cc * LANES, LANES)] = zero16
            return 0

        lax.fori_loop(0, ncr, zero_cnt, 0)

        for i in range(ncr // LANES):
            rowidx[pl.ds(i * LANES, LANES)] = iota16 + i * LANES

        @pl.when(sid < ncr // 8)
        def _():
            pltpu.sync_copy(cnt_v.at[pl.ds(0, 8)],
                            cnt_sh.at[pl.ds(sid * 8, 8)])

        plsc.subcore_barrier()

        ebase = wid * (cpt * K)
        gbase = wid * cpt
        ins = (in0, in1)
        outs = (out0, out1)
        isems = (isem0, isem1)
        wsems = (wsem0, wsem1)
        one16 = jnp.full((LANES,), 1.0, jnp.float32)

        def issue_in(g, p):
            base = ebase + g * K
            pltpu.async_copy(src_hbm.at[pl.ds(base, K)], ins[p].at[0],
                             isems[p])
            pltpu.async_copy(dst_hbm.at[pl.ds(base, K)], ins[p].at[1],
                             isems[p])

        def wait_in(p):
            pltpu.make_async_copy(src_hbm.at[pl.ds(0, K)], ins[p].at[0],
                                  isems[p]).wait()
            pltpu.make_async_copy(dst_hbm.at[pl.ds(0, K)], ins[p].at[1],
                                  isems[p]).wait()

        issue_in(0, 0)
        issue_in(1, 1)

        def half_step(g, p):
            iv = ins[p]
            ov = outs[p]
            wait_in(p)
            @pl.when(g >= 2)
            def _():
                pltpu.make_async_copy(ov, meta_hbm.at[gbase], wsems[p]).wait()

            def group_body(q, _):
                qs = pl.ds(q * LANES, LANES)
                si = iv[0, qs]
                di = iv[1, qs]
                ov[0, qs] = si
                ov[1, qs] = di
                sx = plsc.load_gather(px_v, [si])
                sy = plsc.load_gather(py_v, [si])
                sz = plsc.load_gather(pz_v, [si])
                dx = plsc.load_gather(px_v, [di])
                dy = plsc.load_gather(py_v, [di])
                dz = plsc.load_gather(pz_v, [di])
                rx = dx - sx
                ry = dy - sy
                rz = dz - sz
                v = rx * rx + ry * ry + rz * rz
                norm = v * _rsqrt_newton(v)
                inv = 1.0 / (norm + EPS)
                ov[2, qs] = plsc.bitcast((rx + 1.0) * inv, jnp.int32)
                ov[3, qs] = plsc.bitcast((ry + 1.0) * inv, jnp.int32)
                ov[4, qs] = plsc.bitcast((rz + 1.0) * inv, jnp.int32)
                plsc.addupdate_scatter(cnt_v, [di >> 7, di & 127], one16)
                return 0

            lax.fori_loop(0, K // LANES, group_body, 0)
            pltpu.async_copy(ov, meta_hbm.at[gbase + g], wsems[p])

            @pl.when(g + 2 < cpt)
            def _():
                issue_in(g + 2, p)

        def pair_body(gg, _):
            half_step(2 * gg, 0)
            half_step(2 * gg + 1, 1)
            return 0

        lax.fori_loop(0, cpt // 2, pair_body, 0)
        pltpu.make_async_copy(out0, meta_hbm.at[gbase], wsem0).wait()
        pltpu.make_async_copy(out1, meta_hbm.at[gbase], wsem1).wait()

        pltpu.sync_copy(cnt_v, cnt_sh.at[rowidx], add=True)

        plsc.subcore_barrier()

        @pl.when(sid < ncr // 8)
        def _():
            pltpu.sync_copy(cnt_sh.at[pl.ds(sid * 8, 8)],
                            cnt_hbm.at[cid, pl.ds(sid * 8, 8)])

    return coeff_kernel(src_p, dst_p, posx, posy, posz)


def _sc_scatter_stage(n_pad, cpt, feat, meta, cnt, wx, wy, wz, bsp):
    rps = n_pad // NS
    ncr = n_pad // K
    cpt0 = (2 * cpt * 27 // 40) & ~1
    cpt1 = 2 * cpt - cpt0
    mesh = plsc.VectorSubcoreMesh(core_axis_name="c", subcore_axis_name="s")

    @functools.partial(
        pl.kernel,
        out_type=jax.ShapeDtypeStruct((NC, n_pad, K), jnp.float32),
        mesh=mesh,
        scratch_types=[
            pltpu.VMEM((128,), jnp.float32),
            pltpu.VMEM((128,), jnp.float32),
            pltpu.VMEM((128,), jnp.float32),
            pltpu.VMEM((128,), jnp.float32),
            pltpu.VMEM((5, K), jnp.int32),
            pltpu.VMEM((5, K), jnp.int32),
            pltpu.VMEM((K,), jnp.int32),
            pltpu.VMEM((K,), jnp.int32),
            pltpu.VMEM((K, 128), jnp.float32),
            pltpu.VMEM((K, 128), jnp.float32),
            pltpu.VMEM((ncr, K), jnp.float32),
            pltpu.VMEM_SHARED((n_pad, K), jnp.float32),
            pltpu.SemaphoreType.DMA,
            pltpu.SemaphoreType.DMA,
            pltpu.SemaphoreType.DMA,
            pltpu.SemaphoreType.DMA,
            pltpu.SemaphoreType.DMA,
            pltpu.SemaphoreType.DMA,
        ],
        compiler_params=pltpu.CompilerParams(needs_layout_passes=False),
    )
    def scatter_kernel(feat_hbm, meta_hbm, cnt_hbm, wx_hbm, wy_hbm, wz_hbm,
                       bsp_hbm, out_hbm, wx_v, wy_v, wz_v, bsp_v,
                       meta0, meta1, dstv0, dstv1, rows0, rows1,
                       icnt_v, accum,
                       gsem0, gsem1, ssem0, ssem1, msem0, msem1):
        cid = lax.axis_index("c")
        sid = lax.axis_index("s")

        pltpu.sync_copy(wx_hbm, wx_v)
        pltpu.sync_copy(wy_hbm, wy_v)
        pltpu.sync_copy(wz_hbm, wz_v)
        pltpu.sync_copy(bsp_hbm, bsp_v)

        pltpu.sync_copy(cnt_hbm.at[0], icnt_v)
        one16 = jnp.full((LANES,), 1.0, jnp.float32)

        def inv_body(i, _):
            pltpu.sync_copy(cnt_hbm.at[1, pl.ds(i * 8, 8)],
                            rows0.at[pl.ds(0, 8)])
            for r in range(8):
                for cc in range(K // LANES):
                    s = pl.ds(cc * LANES, LANES)
                    tv = icnt_v[i * 8 + r, s] + rows0[r, s]
                    icnt_v[i * 8 + r, s] = one16 / jnp.maximum(tv, one16)
            return 0

        lax.fori_loop(0, ncr // 8, inv_body, 0)

        zero16 = jnp.zeros((LANES,), jnp.float32)

        def zero_row(r, _):
            for cc in range(128 // LANES):
                rows0[r, pl.ds(cc * LANES, LANES)] = zero16
            return 0

        lax.fori_loop(0, K, zero_row, 0)

        row0 = sid * rps
        for i in range(rps // K):
            pltpu.sync_copy(rows0, accum.at[pl.ds(row0 + i * K, K)])

        plsc.subcore_barrier()

        my_cpt = jnp.where(cid == 0, cpt0, cpt1)
        gbase = jnp.where(cid == 0, sid * cpt0, NS * cpt0 + sid * cpt1)
        metas = (meta0, meta1)
        dstvs = (dstv0, dstv1)
        rows_ = (rows0, rows1)
        gsems = (gsem0, gsem1)
        ssems = (ssem0, ssem1)
        msems = (msem0, msem1)

        HK = K // 2

        def issue_gather(mt, rw, sem):
            pltpu.async_copy(feat_hbm.at[mt.at[0, pl.ds(0, HK)]],
                             rw.at[pl.ds(0, HK)], sem)
            pltpu.async_copy(feat_hbm.at[mt.at[0, pl.ds(HK, HK)]],
                             rw.at[pl.ds(HK, HK)], sem)

        def wait_gather(mt, rw, sem):
            pltpu.make_async_copy(feat_hbm.at[mt.at[0, pl.ds(0, HK)]],
                                  rw.at[pl.ds(0, HK)], sem).wait()
            pltpu.make_async_copy(feat_hbm.at[mt.at[0, pl.ds(HK, HK)]],
                                  rw.at[pl.ds(HK, HK)], sem).wait()

        pltpu.sync_copy(meta_hbm.at[gbase], meta0)
        issue_gather(meta0, rows0, gsem0)
        pltpu.async_copy(meta_hbm.at[gbase + 1], meta1, msem1)

        wxs = tuple(wx_v[pl.ds(c * LANES, LANES)] for c in range(8))
        wys = tuple(wy_v[pl.ds(c * LANES, LANES)] for c in range(8))
        wzs = tuple(wz_v[pl.ds(c * LANES, LANES)] for c in range(8))

        def compute_chunk(mt, dv, rw):
            def group_body(q, _):
                qs = pl.ds(q * LANES, LANES)
                di = mt[1, qs]
                dv[qs] = di
                ic = plsc.load_gather(icnt_v, [di >> 7, di & 127])
                aq = plsc.bitcast(mt[2, qs], jnp.float32)
                bq = plsc.bitcast(mt[3, qs], jnp.float32)
                cq = plsc.bitcast(mt[4, qs], jnp.float32)
                for l in range(LANES):
                    a = jnp.full((LANES,), aq[l])
                    b = jnp.full((LANES,), bq[l])
                    c = jnp.full((LANES,), cq[l])
                    icb = jnp.full((LANES,), ic[l])
                    j = q * LANES + l
                    for c8 in range(128 // LANES):
                        s = pl.ds(c8 * LANES, LANES)
                        z = (a * wxs[c8] + b * wys[c8] + c * wzs[c8]
                             + bsp_v[s])
                        e = icb * jnp.maximum(z, 0.01 * z)
                        rw[j, s] = e * rw[j, s]
                return 0

            lax.fori_loop(0, K // LANES, group_body, 0)

        def half_step(g, p):
            q = 1 - p
            mt_p, mt_q = metas[p], metas[q]
            rw_p, rw_q = rows_[p], rows_[q]
            wait_gather(mt_p, rw_p, gsems[p])

            @pl.when(g + 1 < my_cpt)
            def _():
                pltpu.make_async_copy(meta_hbm.at[gbase], mt_q,
                                      msems[q]).wait()
                @pl.when(g >= 1)
                def _():
                    pltpu.make_async_copy(rw_q, accum.at[dstvs[q]],
                                          ssems[q]).wait()
                issue_gather(mt_q, rw_q, gsems[q])

            compute_chunk(mt_p, dstvs[p], rw_p)
            pltpu.async_copy(rw_p, accum.at[dstvs[p]], ssems[p], add=True)

            @pl.when(g + 2 < my_cpt)
            def _():
                pltpu.async_copy(meta_hbm.at[gbase + g + 2], mt_p, msems[p])

        def pair_body(gg, _):
            half_step(2 * gg, 0)
            half_step(2 * gg + 1, 1)
            return 0

        lax.fori_loop(0, my_cpt // 2, pair_body, 0)

        pltpu.make_async_copy(rows0, accum.at[dstv0], ssem0).wait()
        pltpu.make_async_copy(rows1, accum.at[dstv1], ssem1).wait()

        plsc.subcore_barrier()
        pltpu.sync_copy(accum.at[pl.ds(row0, rps)],
                        out_hbm.at[cid, pl.ds(row0, rps)])

    return scatter_kernel(feat, meta, cnt, wx, wy, wz, bsp)


def _tc_body(feat_ref, acc_ref, ws_ref, wn_ref, b3_ref, out_ref):
    h_mean = acc_ref[0] + acc_ref[1]
    dn = (((1,), (1,)), ((), ()))
    t = lax.dot_general(feat_ref[...], ws_ref[...], dn,
                        precision=lax.Precision.HIGHEST,
                        preferred_element_type=jnp.float32)
    t = t + lax.dot_general(h_mean, wn_ref[...], dn,
                            precision=lax.Precision.HIGHEST,
                            preferred_element_type=jnp.float32)
    t = t + (b3_ref[0] + b3_ref[1] + b3_ref[2])[None, :]
    out_ref[...] = jnp.maximum(t, 0.01 * t)


def kernel(feat, edge_index, position, W_self, b_self, W_spatial, b_spatial,
           W_neigh, b_neigh, bias):
    n, f = feat.shape
    e = edge_index.shape[1]

    nw = NC * NS
    cpt = -(-e // (nw * K))
    cpt = cpt + (cpt % 2)
    e_pad = nw * cpt * K
    src_p = jnp.concatenate(
        [edge_index[0], jnp.zeros((e_pad - e,), edge_index.dtype)]).astype(jnp.int32)
    dst_p = jnp.concatenate(
        [edge_index[1], jnp.full((e_pad - e,), n, edge_index.dtype)]).astype(jnp.int32)
    n_pad = -(-(n + 1) // (NS * K)) * (NS * K)
    ncr = n_pad // K
    posx = position[:, 0]
    posy = position[:, 1]
    posz = position[:, 2]
    wx = W_spatial[:, 0]
    wy = W_spatial[:, 1]
    wz = W_spatial[:, 2]

    meta, cnt = _sc_coeff_stage(ncr, cpt, src_p, dst_p, posx, posy, posz)
    acc = _sc_scatter_stage(n_pad, cpt, feat, meta, cnt, wx, wy, wz,
                            b_spatial)

    b3 = jnp.stack([b_self, b_neigh, bias])
    blk = 1000
    grid = n // blk
    return pl.pallas_call(
        _tc_body,
        grid=(grid,),
        in_specs=[
            pl.BlockSpec((blk, f), lambda i: (i, 0)),
            pl.BlockSpec((NC, blk, f), lambda i: (0, i, 0)),
            pl.BlockSpec((f, f), lambda i: (0, 0)),
            pl.BlockSpec((f, f), lambda i: (0, 0)),
            pl.BlockSpec((3, f), lambda i: (0, 0)),
        ],
        out_specs=pl.BlockSpec((blk, f), lambda i: (i, 0)),
        out_shape=jax.ShapeDtypeStruct((n, f), jnp.float32),
    )(feat, acc[:, :n, :], W_self, W_neigh, b3)


# speedup vs baseline: 2.0985x; 2.0985x over previous
import functools

import jax
import jax.numpy as jnp
from jax import lax
from jax.experimental import pallas as pl
from jax.experimental.pallas import tpu as pltpu
from jax.experimental.pallas import tpu_sc as plsc

EPS = 1e-07
NC = 2
NS = 16
LANES = 16
K = 128


def _rsqrt_newton(v):
    i = plsc.bitcast(v, jnp.int32)
    y = plsc.bitcast(jnp.int32(0x5F3759DF) - (i >> 1), jnp.float32)
    for _ in range(3):
        y = y * (1.5 - 0.5 * v * y * y)
    return y


def _sc_coeff_stage(ncr, cpt, src_p, dst_p, posx, posy, posz):
    n = posx.shape[0]
    g_total = NC * NS * cpt
    mesh = plsc.VectorSubcoreMesh(core_axis_name="c", subcore_axis_name="s")

    @functools.partial(
        pl.kernel,
        out_type=(
            jax.ShapeDtypeStruct((g_total, 5, K), jnp.int32),
            jax.ShapeDtypeStruct((NC, ncr, K), jnp.float32),
        ),
        mesh=mesh,
        scratch_types=[
            pltpu.VMEM((n,), jnp.float32),
            pltpu.VMEM((n,), jnp.float32),
            pltpu.VMEM((n,), jnp.float32),
            pltpu.VMEM((2, K), jnp.int32),
            pltpu.VMEM((2, K), jnp.int32),
            pltpu.VMEM((5, K), jnp.int32),
            pltpu.VMEM((5, K), jnp.int32),
            pltpu.VMEM((ncr, K), jnp.float32),
            pltpu.VMEM((ncr,), jnp.int32),
            pltpu.VMEM_SHARED((ncr, K), jnp.float32),
            pltpu.SemaphoreType.DMA,
            pltpu.SemaphoreType.DMA,
            pltpu.SemaphoreType.DMA,
            pltpu.SemaphoreType.DMA,
        ],
        compiler_params=pltpu.CompilerParams(needs_layout_passes=False),
    )
    def coeff_kernel(src_hbm, dst_hbm, px_hbm, py_hbm, pz_hbm,
                     meta_hbm, cnt_hbm,
                     px_v, py_v, pz_v, in0, in1, out0, out1,
                     cnt_v, rowidx, cnt_sh, isem0, isem1, wsem0, wsem1):
        cid = lax.axis_index("c")
        sid = lax.axis_index("s")
        wid = cid * NS + sid

        pltpu.sync_copy(px_hbm, px_v)
        pltpu.sync_copy(py_hbm, py_v)
        pltpu.sync_copy(pz_hbm, pz_v)

        zero16 = jnp.zeros((LANES,), jnp.float32)
        iota16 = lax.iota(jnp.int32, LANES)

        def zero_cnt(r, _):
            for cc in range(K // LANES):
                cnt_v[r, pl.ds(cc * LANES, LANES)] = zero16
            return 0

        lax.fori_loop(0, ncr, zero_cnt, 0)

        for i in range(ncr // LANES):
            rowidx[pl.ds(i * LANES, LANES)] = iota16 + i * LANES

        @pl.when(sid < ncr // 8)
        def _():
            pltpu.sync_copy(cnt_v.at[pl.ds(0, 8)],
                            cnt_sh.at[pl.ds(sid * 8, 8)])

        plsc.subcore_barrier()

        ebase = wid * (cpt * K)
        gbase = wid * cpt
        ins = (in0, in1)
        outs = (out0, out1)
        isems = (isem0, isem1)
        wsems = (wsem0, wsem1)
        one16 = jnp.full((LANES,), 1.0, jnp.float32)

        def issue_in(g, p):
            base = ebase + g * K
            pltpu.async_copy(src_hbm.at[pl.ds(base, K)], ins[p].at[0],
                             isems[p])
            pltpu.async_copy(dst_hbm.at[pl.ds(base, K)], ins[p].at[1],
                             isems[p])

        def wait_in(p):
            pltpu.make_async_copy(src_hbm.at[pl.ds(0, K)], ins[p].at[0],
                                  isems[p]).wait()
            pltpu.make_async_copy(dst_hbm.at[pl.ds(0, K)], ins[p].at[1],
                                  isems[p]).wait()

        issue_in(0, 0)
        issue_in(1, 1)

        def half_step(g, p):
            iv = ins[p]
            ov = outs[p]
            wait_in(p)
            @pl.when(g >= 2)
            def _():
                pltpu.make_async_copy(ov, meta_hbm.at[gbase], wsems[p]).wait()

            def group_body(q, _):
                qs = pl.ds(q * LANES, LANES)
                si = iv[0, qs]
                di = iv[1, qs]
                ov[0, qs] = si
                ov[1, qs] = di
                sx = plsc.load_gather(px_v, [si])
                sy = plsc.load_gather(py_v, [si])
                sz = plsc.load_gather(pz_v, [si])
                dx = plsc.load_gather(px_v, [di])
                dy = plsc.load_gather(py_v, [di])
                dz = plsc.load_gather(pz_v, [di])
                rx = dx - sx
                ry = dy - sy
                rz = dz - sz
                v = rx * rx + ry * ry + rz * rz
                norm = v * _rsqrt_newton(v)
                inv = 1.0 / (norm + EPS)
                ov[2, qs] = plsc.bitcast((rx + 1.0) * inv, jnp.int32)
                ov[3, qs] = plsc.bitcast((ry + 1.0) * inv, jnp.int32)
                ov[4, qs] = plsc.bitcast((rz + 1.0) * inv, jnp.int32)
                plsc.addupdate_scatter(cnt_v, [di >> 7, di & 127], one16)
                return 0

            lax.fori_loop(0, K // LANES, group_body, 0)
            pltpu.async_copy(ov, meta_hbm.at[gbase + g], wsems[p])

            @pl.when(g + 2 < cpt)
            def _():
                issue_in(g + 2, p)

        def pair_body(gg, _):
            half_step(2 * gg, 0)
            half_step(2 * gg + 1, 1)
            return 0

        lax.fori_loop(0, cpt // 2, pair_body, 0)
        pltpu.make_async_copy(out0, meta_hbm.at[gbase], wsem0).wait()
        pltpu.make_async_copy(out1, meta_hbm.at[gbase], wsem1).wait()

        pltpu.sync_copy(cnt_v, cnt_sh.at[rowidx], add=True)

        plsc.subcore_barrier()

        @pl.when(sid < ncr // 8)
        def _():
            pltpu.sync_copy(cnt_sh.at[pl.ds(sid * 8, 8)],
                            cnt_hbm.at[cid, pl.ds(sid * 8, 8)])

    return coeff_kernel(src_p, dst_p, posx, posy, posz)


def _sc_scatter_stage(n_pad, cpt, feat, meta, cnt, wx, wy, wz, bsp):
    rps = n_pad // NS
    ncr = n_pad // K
    cpt0 = (2 * cpt * 27 // 40) & ~1
    cpt1 = 2 * cpt - cpt0
    mesh = plsc.VectorSubcoreMesh(core_axis_name="c", subcore_axis_name="s")

    @functools.partial(
        pl.kernel,
        out_type=jax.ShapeDtypeStruct((NC, n_pad, K), jnp.float32),
        mesh=mesh,
        scratch_types=[
            pltpu.VMEM((128,), jnp.float32),
            pltpu.VMEM((128,), jnp.float32),
            pltpu.VMEM((128,), jnp.float32),
            pltpu.VMEM((128,), jnp.float32),
            pltpu.VMEM((5, K), jnp.int32),
            pltpu.VMEM((5, K), jnp.int32),
            pltpu.VMEM((K,), jnp.int32),
            pltpu.VMEM((K,), jnp.int32),
            pltpu.VMEM((K, 128), jnp.float32),
            pltpu.VMEM((K, 128), jnp.float32),
            pltpu.VMEM((2, 8, K), jnp.float32),
            pltpu.VMEM_SHARED((n_pad, K), jnp.float32),
            pltpu.SemaphoreType.DMA,
            pltpu.SemaphoreType.DMA,
            pltpu.SemaphoreType.DMA,
            pltpu.SemaphoreType.DMA,
            pltpu.SemaphoreType.DMA,
            pltpu.SemaphoreType.DMA,
        ],
        compiler_params=pltpu.CompilerParams(needs_layout_passes=False),
    )
    def scatter_kernel(feat_hbm, meta_hbm, cnt_hbm, wx_hbm, wy_hbm, wz_hbm,
                       bsp_hbm, out_hbm, wx_v, wy_v, wz_v, bsp_v,
                       meta0, meta1, dstv0, dstv1, rows0, rows1,
                       cw_v, accum,
                       gsem0, gsem1, ssem0, ssem1, msem0, msem1):
        cid = lax.axis_index("c")
        sid = lax.axis_index("s")

        pltpu.sync_copy(wx_hbm, wx_v)
        pltpu.sync_copy(wy_hbm, wy_v)
        pltpu.sync_copy(wz_hbm, wz_v)
        pltpu.sync_copy(bsp_hbm, bsp_v)

        zero16 = jnp.zeros((LANES,), jnp.float32)

        def zero_row(r, _):
            for cc in range(128 // LANES):
                rows0[r, pl.ds(cc * LANES, LANES)] = zero16
            return 0

        lax.fori_loop(0, K, zero_row, 0)

        row0 = sid * rps
        for i in range(rps // K):
            pltpu.sync_copy(rows0, accum.at[pl.ds(row0 + i * K, K)])

        plsc.subcore_barrier()

        my_cpt = jnp.where(cid == 0, cpt0, cpt1)
        gbase = jnp.where(cid == 0, sid * cpt0, NS * cpt0 + sid * cpt1)
        metas = (meta0, meta1)
        dstvs = (dstv0, dstv1)
        rows_ = (rows0, rows1)
        gsems = (gsem0, gsem1)
        ssems = (ssem0, ssem1)
        msems = (msem0, msem1)

        HK = K // 2

        def issue_gather(mt, rw, sem):
            pltpu.async_copy(feat_hbm.at[mt.at[0, pl.ds(0, HK)]],
                             rw.at[pl.ds(0, HK)], sem)
            pltpu.async_copy(feat_hbm.at[mt.at[0, pl.ds(HK, HK)]],
                             rw.at[pl.ds(HK, HK)], sem)

        def wait_gather(mt, rw, sem):
            pltpu.make_async_copy(feat_hbm.at[mt.at[0, pl.ds(0, HK)]],
                                  rw.at[pl.ds(0, HK)], sem).wait()
            pltpu.make_async_copy(feat_hbm.at[mt.at[0, pl.ds(HK, HK)]],
                                  rw.at[pl.ds(HK, HK)], sem).wait()

        pltpu.sync_copy(meta_hbm.at[gbase], meta0)
        issue_gather(meta0, rows0, gsem0)
        pltpu.async_copy(meta_hbm.at[gbase + 1], meta1, msem1)

        wxs = tuple(wx_v[pl.ds(c * LANES, LANES)] for c in range(8))
        wys = tuple(wy_v[pl.ds(c * LANES, LANES)] for c in range(8))
        wzs = tuple(wz_v[pl.ds(c * LANES, LANES)] for c in range(8))
        bsps = tuple(bsp_v[pl.ds(c * LANES, LANES)] for c in range(8))

        def compute_chunk(mt, dv, rw):
            def group_body(q, _):
                qs = pl.ds(q * LANES, LANES)
                dv[qs] = mt[1, qs]
                aq = plsc.bitcast(mt[2, qs], jnp.float32)
                bq = plsc.bitcast(mt[3, qs], jnp.float32)
                cq = plsc.bitcast(mt[4, qs], jnp.float32)
                for l in range(LANES):
                    a = jnp.full((LANES,), aq[l])
                    b = jnp.full((LANES,), bq[l])
                    c = jnp.full((LANES,), cq[l])
                    j = q * LANES + l
                    for c8 in range(128 // LANES):
                        s = pl.ds(c8 * LANES, LANES)
                        z = a * wxs[c8] + b * wys[c8] + c * wzs[c8] + bsps[c8]
                        e = jnp.maximum(z, 0.01 * z)
                        rw[j, s] = e * rw[j, s]
                return 0

            lax.fori_loop(0, K // LANES, group_body, 0)

        def half_step(g, p):
            q = 1 - p
            mt_p, mt_q = metas[p], metas[q]
            rw_p, rw_q = rows_[p], rows_[q]
            wait_gather(mt_p, rw_p, gsems[p])

            @pl.when(g + 1 < my_cpt)
            def _():
                pltpu.make_async_copy(meta_hbm.at[gbase], mt_q,
                                      msems[q]).wait()
                @pl.when(g >= 1)
                def _():
                    pltpu.make_async_copy(rw_q, accum.at[dstvs[q]],
                                          ssems[q]).wait()
                issue_gather(mt_q, rw_q, gsems[q])

            compute_chunk(mt_p, dstvs[p], rw_p)
            pltpu.async_copy(rw_p, accum.at[dstvs[p]], ssems[p], add=True)

            @pl.when(g + 2 < my_cpt)
            def _():
                pltpu.async_copy(meta_hbm.at[gbase + g + 2], mt_p, msems[p])

        def pair_body(gg, _):
            half_step(2 * gg, 0)
            half_step(2 * gg + 1, 1)
            return 0

        lax.fori_loop(0, my_cpt // 2, pair_body, 0)

        pltpu.make_async_copy(rows0, accum.at[dstv0], ssem0).wait()
        pltpu.make_async_copy(rows1, accum.at[dstv1], ssem1).wait()

        plsc.subcore_barrier()

        one16 = jnp.full((LANES,), 1.0, jnp.float32)

        @pl.when(sid < ncr // 8)
        def _():
            pltpu.sync_copy(cnt_hbm.at[0, pl.ds(sid * 8, 8)], cw_v.at[0])
            pltpu.sync_copy(cnt_hbm.at[1, pl.ds(sid * 8, 8)], cw_v.at[1])

            def blk_body(blk, _):
                rbase = (sid * 8 + blk) * K
                pltpu.sync_copy(accum.at[pl.ds(rbase, K)], rows0)

                def scale_body(u, _):
                    us = pl.ds(u * LANES, LANES)
                    iv = one16 / jnp.maximum(
                        cw_v[0, blk, us] + cw_v[1, blk, us], one16)
                    for l in range(LANES):
                        r = u * LANES + l
                        ib = jnp.full((LANES,), iv[l])
                        for cc in range(K // LANES):
                            s = pl.ds(cc * LANES, LANES)
                            rows0[r, s] = ib * rows0[r, s]
                    return 0

                lax.fori_loop(0, K // LANES, scale_body, 0)
                pltpu.sync_copy(rows0, out_hbm.at[cid, pl.ds(rbase, K)])
                return 0

            lax.fori_loop(0, 8, blk_body, 0)

    return scatter_kernel(feat, meta, cnt, wx, wy, wz, bsp)


def _tc_body(feat_ref, acc_ref, ws_ref, wn_ref, b3_ref, out_ref):
    h_mean = acc_ref[0] + acc_ref[1]
    dn = (((1,), (1,)), ((), ()))
    t = lax.dot_general(feat_ref[...], ws_ref[...], dn,
                        precision=lax.Precision.HIGHEST,
                        preferred_element_type=jnp.float32)
    t = t + lax.dot_general(h_mean, wn_ref[...], dn,
                            precision=lax.Precision.HIGHEST,
                            preferred_element_type=jnp.float32)
    t = t + (b3_ref[0] + b3_ref[1] + b3_ref[2])[None, :]
    out_ref[...] = jnp.maximum(t, 0.01 * t)


def kernel(feat, edge_index, position, W_self, b_self, W_spatial, b_spatial,
           W_neigh, b_neigh, bias):
    n, f = feat.shape
    e = edge_index.shape[1]

    nw = NC * NS
    cpt = -(-e // (nw * K))
    cpt = cpt + (cpt % 2)
    e_pad = nw * cpt * K
    src_p = jnp.concatenate(
        [edge_index[0], jnp.zeros((e_pad - e,), edge_index.dtype)]).astype(jnp.int32)
    dst_p = jnp.concatenate(
        [edge_index[1], jnp.full((e_pad - e,), n, edge_index.dtype)]).astype(jnp.int32)
    n_pad = -(-(n + 1) // (NS * K)) * (NS * K)
    ncr = n_pad // K
    posx = position[:, 0]
    posy = position[:, 1]
    posz = position[:, 2]
    wx = W_spatial[:, 0]
    wy = W_spatial[:, 1]
    wz = W_spatial[:, 2]

    meta, cnt = _sc_coeff_stage(ncr, cpt, src_p, dst_p, posx, posy, posz)
    acc = _sc_scatter_stage(n_pad, cpt, feat, meta, cnt, wx, wy, wz,
                            b_spatial)

    b3 = jnp.stack([b_self, b_neigh, bias])
    blk = 1000
    grid = n // blk
    return pl.pallas_call(
        _tc_body,
        grid=(grid,),
        in_specs=[
            pl.BlockSpec((blk, f), lambda i: (i, 0)),
            pl.BlockSpec((NC, blk, f), lambda i: (0, i, 0)),
            pl.BlockSpec((f, f), lambda i: (0, 0)),
            pl.BlockSpec((f, f), lambda i: (0, 0)),
            pl.BlockSpec((3, f), lambda i: (0, 0)),
        ],
        out_specs=pl.BlockSpec((blk, f), lambda i: (i, 0)),
        out_shape=jax.ShapeDtypeStruct((n, f), jnp.float32),
    )(feat, acc[:, :n, :], W_self, W_neigh, b3)
